# baseline (speedup 1.0000x reference)
import functools

import jax
import jax.numpy as jnp
from jax import lax
from jax.experimental import pallas as pl
from jax.experimental.pallas import tpu as pltpu
from jax.experimental.pallas import tpu_sc as plsc

NUM_FRAMES = 32
T = 300
ROW = 3 * 224 * 224
NCHUNK = 4
CH = ROW // NCHUNK

_info = plsc.get_sparse_core_info()
_NC, _NS = _info.num_cores, _info.num_subcores


def _clip_sampler_kernel(frames_hbm, out_hbm, sem):
    wid = lax.axis_index("s") * _NC + lax.axis_index("c")
    src = (wid * (T - 1)) // (NUM_FRAMES - 1)
    cp = pltpu.make_async_copy(frames_hbm.at[src], out_hbm.at[wid], sem)
    cp.start()
    cp.wait()


@jax.jit
def kernel(frames):
    frames3 = frames.reshape(T, NCHUNK, CH)
    mesh = plsc.VectorSubcoreMesh(core_axis_name="c", subcore_axis_name="s")
    out = pl.kernel(
        _clip_sampler_kernel,
        out_type=jax.ShapeDtypeStruct((NUM_FRAMES, NCHUNK, CH), jnp.float32),
        mesh=mesh,
        scratch_types=[
            pltpu.SemaphoreType.DMA,
        ],
    )(frames3)
    return out.reshape(NUM_FRAMES, 3, 224, 224)


# speedup vs baseline: 24.8458x; 24.8458x over previous
import functools

import jax
import jax.numpy as jnp
from jax import lax
from jax.experimental import pallas as pl
from jax.experimental.pallas import tpu as pltpu
from jax.experimental.pallas import tpu_sc as plsc

NUM_FRAMES = 32
T = 300
NCHUNK = 3

_info = plsc.get_sparse_core_info()
_NC, _NS = _info.num_cores, _info.num_subcores


def _clip_sampler_kernel(
    frames_hbm, out_hbm, buf_a, buf_b, sin_a, sin_b, sout_a, sout_b
):
    wid = lax.axis_index("s") * _NC + lax.axis_index("c")
    src = (wid * (T - 1)) // (NUM_FRAMES - 1)

    bufs = (buf_a, buf_b)
    sins = (sin_a, sin_b)
    souts = (sout_a, sout_b)

    def in_copy(c):
        return pltpu.make_async_copy(frames_hbm.at[src, c], bufs[c % 2], sins[c % 2])

    def out_copy(c):
        return pltpu.make_async_copy(bufs[c % 2], out_hbm.at[wid, c], souts[c % 2])

    in_copy(0).start()
    in_copy(1).start()
    for c in range(NCHUNK):
        in_copy(c).wait()
        out_copy(c).start()
        if c + 2 < NCHUNK:
            out_copy(c).wait()
            in_copy(c + 2).start()
    out_copy(NCHUNK - 2).wait()
    out_copy(NCHUNK - 1).wait()


@jax.jit
def kernel(frames):
    mesh = plsc.VectorSubcoreMesh(core_axis_name="c", subcore_axis_name="s")
    return pl.kernel(
        _clip_sampler_kernel,
        out_type=jax.ShapeDtypeStruct((NUM_FRAMES, 3, 224, 224), jnp.float32),
        mesh=mesh,
        scratch_types=[
            pltpu.VMEM((224, 224), jnp.float32),
            pltpu.VMEM((224, 224), jnp.float32),
            pltpu.SemaphoreType.DMA,
            pltpu.SemaphoreType.DMA,
            pltpu.SemaphoreType.DMA,
            pltpu.SemaphoreType.DMA,
        ],
    )(frames)


# speedup vs baseline: 25.0439x; 1.0080x over previous
import functools

import jax
import jax.numpy as jnp
from jax import lax
from jax.experimental import pallas as pl
from jax.experimental.pallas import tpu as pltpu
from jax.experimental.pallas import tpu_sc as plsc

NUM_FRAMES = 32
T = 300
CROWS = 56
CPP = 224 // CROWS
NCHUNK = 3 * CPP
NBUF = 8

_info = plsc.get_sparse_core_info()
_NC, _NS = _info.num_cores, _info.num_subcores


def _clip_sampler_kernel(frames_hbm, out_hbm, *scratch):
    bufs = scratch[:NBUF]
    sins = scratch[NBUF:2 * NBUF]
    souts = scratch[2 * NBUF:]

    wid = lax.axis_index("s") * _NC + lax.axis_index("c")
    src = (wid * (T - 1)) // (NUM_FRAMES - 1)

    def in_copy(c):
        ch, r = c // CPP, (c % CPP) * CROWS
        return pltpu.make_async_copy(
            frames_hbm.at[src, ch, pl.ds(r, CROWS)], bufs[c % NBUF], sins[c % NBUF]
        )

    def out_copy(c):
        ch, r = c // CPP, (c % CPP) * CROWS
        return pltpu.make_async_copy(
            bufs[c % NBUF], out_hbm.at[wid, ch, pl.ds(r, CROWS)], souts[c % NBUF]
        )

    for c in range(min(NBUF, NCHUNK)):
        in_copy(c).start()
    for c in range(NCHUNK):
        in_copy(c).wait()
        out_copy(c).start()
        if c + NBUF < NCHUNK:
            out_copy(c).wait()
            in_copy(c + NBUF).start()
    for c in range(max(0, NCHUNK - NBUF), NCHUNK):
        out_copy(c).wait()


@jax.jit
def kernel(frames):
    mesh = plsc.VectorSubcoreMesh(core_axis_name="c", subcore_axis_name="s")
    return pl.kernel(
        _clip_sampler_kernel,
        out_type=jax.ShapeDtypeStruct((NUM_FRAMES, 3, 224, 224), jnp.float32),
        mesh=mesh,
        scratch_types=(
            [pltpu.VMEM((CROWS, 224), jnp.float32)] * NBUF
            + [pltpu.SemaphoreType.DMA] * (2 * NBUF)
        ),
    )(frames)
